# baseline (speedup 1.0000x reference)
import jax
import jax.numpy as jnp
from jax import lax
from jax.experimental import pallas as pl
from jax.experimental.pallas import tpu as pltpu
from jax.experimental.pallas import tpu_sc as plsc

B = 16
L = 4096
CH = 16
SEG = L // 2
NCH = SEG // CH
PAD = 8
OUTW = 16


def _body(x_hbm, out_hbm, buf, tab, stage, stage2, res, shared):
    cc = lax.axis_index("c")
    s = lax.axis_index("s")
    row = cc * 8 + (s // 2)
    h = s & 1

    zeros16 = jnp.zeros((CH,), jnp.float32)
    buf[pl.ds(0, CH)] = zeros16
    iota = lax.iota(jnp.int32, CH)
    iota16 = iota * 16
    zi = jnp.zeros((CH,), jnp.int32)
    tab[pl.ds(0, CH)] = jnp.where(iota == 0, 1,
                        jnp.where(iota <= 6, 1 << 10,
                        jnp.where(iota >= 10, 1 << 20, 0)))

    @pl.when(h == 0)
    def _():
        pltpu.sync_copy(x_hbm.at[row, pl.ds(0, SEG)], buf.at[pl.ds(PAD, SEG)])

    @pl.when(h == 1)
    def _():
        pltpu.sync_copy(x_hbm.at[row], buf.at[pl.ds(PAD, L)])

    def bs_cond(st):
        return (st[1] == 0) & (st[2] >= 0)

    def bs_body(st):
        l1, l2, cb = st
        xb = buf[pl.ds(PAD + cb * CH, CH)]
        vb = xb.astype(jnp.int32)
        keyb = jnp.where(vb != 0, cb * 256 + 16 + iota16 + vb, 0)
        m1 = jnp.max(keyb)
        m2 = jnp.max(jnp.where(keyb == m1, 0, keyb))
        l1n = jnp.where(l1 == 0, m1, l1)
        l2n = jnp.where(l1 == 0, m2, jnp.where(l2 == 0, m1, l2))
        return (l1n, l2n, cb - 1)

    l1, l2, _ = lax.while_loop(
        bs_cond, bs_body,
        (jnp.int32(0), jnp.int32(0),
         jnp.where(h == 1, NCH - 1, -1).astype(jnp.int32)))

    @pl.when(h == 1)
    def _():
        w = buf[pl.ds(PAD + SEG - CH, CH)]
        buf[pl.ds(PAD + SEG - CH, CH)] = jnp.where(iota == 15, 0.0, w)

    ebase = h * SEG
    kbase = ebase * 16
    init = (l1, l2, jnp.int32(0), zi, zi)

    @plsc.parallel_loop(0, NCH, 1, unroll=2, carry=init)
    def fin(c, carry):
        carryE, carryO, cpar, acc1, acc2 = carry
        base = PAD + ebase + c * CH
        xc = buf[pl.ds(base, CH)]
        xp = buf[pl.ds(base - 1, CH)]
        vp = xp.astype(jnp.int32)
        mp = vp != 0
        mpi = jnp.where(mp, 1, 0)
        cs = plsc.cumsum(mpi)
        rank = cs + cpar
        par_even = (rank & 1) == 0
        kraw = kbase + c * 256 + iota16 + vp
        keyp = jnp.where(mp, kraw, 0)
        keyE = jnp.where(par_even, keyp, zi)
        keyO = jnp.where(par_even, zi, keyp)
        cumE = plsc.cummax(keyE)
        cumO = plsc.cummax(keyO)
        ME = jnp.maximum(cumE, carryE)
        MO = jnp.maximum(cumO, carryO)
        p1k = jnp.maximum(ME, MO)
        p2k = jnp.minimum(ME, MO)
        vc = xc.astype(jnp.int32)
        mc = xc != 0.0
        mykey = kbase + c * 256 + 16 + iota16 + vc
        d1 = (mykey - p1k) & 15
        d2 = (mykey - p2k) & 15
        a1 = mc & (p1k > 0)
        t1 = plsc.load_gather(tab, [d1])
        acc1 = acc1 + jnp.where(a1, t1, 0)
        hit2 = mc & (p2k > 0) & (d2 == 0)
        acc2 = acc2 + jnp.where(hit2, 1, 0) + jnp.where(mc, 1 << 16, 0)
        return (jnp.maximum(carryE, cumE[15]),
                jnp.maximum(carryO, cumO[15]),
                (cpar + cs[15]) & 1,
                acc1, acc2)

    stage[pl.ds(0, CH)] = fin[3]
    stage[pl.ds(CH, CH)] = fin[4]
    pltpu.sync_copy(stage, shared.at[s])
    plsc.subcore_barrier()

    @pl.when(h == 0)
    def _():
        pltpu.sync_copy(shared.at[s + 1], stage2)
        acc1 = fin[3] + stage2[pl.ds(0, CH)]
        acc2 = fin[4] + stage2[pl.ds(CH, CH)]
        rep = jnp.sum(acc1 & 1023).astype(jnp.float32)
        inc = jnp.sum((acc1 >> 10) & 1023).astype(jnp.float32)
        dec = jnp.sum(acc1 >> 20).astype(jnp.float32)
        p2 = jnp.sum(acc2 & 0xFFFF).astype(jnp.float32)
        cnt = jnp.sum(acc2 >> 16)
        cf = cnt.astype(jnp.float32)
        den1 = jnp.maximum(cf - 1.0, 1.0)
        den2 = jnp.maximum(cf - 2.0, 1.0)
        num = jnp.where(iota == 0, rep,
              jnp.where(iota == 1, inc,
              jnp.where(iota == 2, dec,
              jnp.where(iota == 3, p2, 0.0))))
        den = jnp.where(iota == 3, den2, den1)
        gate = jnp.where(iota < 3, cnt > 1, cnt >= 4) & (iota < 4)
        res[pl.ds(0, CH)] = jnp.where(gate, num / den, 0.0)
        pltpu.sync_copy(res.at[pl.ds(0, 4)], out_hbm.at[row])


@jax.jit
def kernel(x):
    run = pl.kernel(
        _body,
        out_type=jax.ShapeDtypeStruct((B, 4), jnp.float32),
        mesh=plsc.VectorSubcoreMesh(core_axis_name="c", subcore_axis_name="s"),
        scratch_types=[
            pltpu.VMEM((PAD + L,), jnp.float32),
            pltpu.VMEM((CH,), jnp.int32),
            pltpu.VMEM((2 * CH,), jnp.int32),
            pltpu.VMEM((2 * CH,), jnp.int32),
            pltpu.VMEM((OUTW,), jnp.float32),
            pltpu.VMEM_SHARED((16, 2 * CH), jnp.int32),
        ],
        compiler_params=pltpu.CompilerParams(
            needs_layout_passes=False, use_tc_tiling_on_sc=False,
            skip_device_barrier=True),
    )
    return run(x)


# speedup vs baseline: 1.0754x; 1.0754x over previous
import jax
import jax.numpy as jnp
from jax import lax
from jax.experimental import pallas as pl
from jax.experimental.pallas import tpu as pltpu
from jax.experimental.pallas import tpu_sc as plsc

B = 16
L = 4096
CH = 16
SEG = L // 2
NCH = SEG // CH
PAD = 8
OUTW = 16


def _body(x_hbm, out_hbm, buf, tab, stage, stage2, res, shared):
    cc = lax.axis_index("c")
    s = lax.axis_index("s")
    row = cc * 8 + (s // 2)
    h = s & 1

    zeros16 = jnp.zeros((CH,), jnp.float32)
    buf[pl.ds(0, CH)] = zeros16
    iota = lax.iota(jnp.int32, CH)
    iota16 = iota * 16
    zi = jnp.zeros((CH,), jnp.int32)
    tab[pl.ds(0, CH)] = jnp.where(iota == 0, 1,
                        jnp.where(iota <= 6, 1 << 10,
                        jnp.where(iota >= 10, 1 << 20, 0)))

    @pl.when(h == 0)
    def _():
        pltpu.sync_copy(x_hbm.at[row, pl.ds(0, SEG)], buf.at[pl.ds(PAD, SEG)])

    @pl.when(h == 1)
    def _():
        pltpu.sync_copy(x_hbm.at[row], buf.at[pl.ds(PAD, L)])

    def bs_cond(st):
        return (st[1] == 0) & (st[2] >= 0)

    def bs_body(st):
        l1, l2, cb = st
        xb = buf[pl.ds(PAD + cb * CH, CH)]
        vb = xb.astype(jnp.int32)
        keyb = jnp.where(vb != 0, cb * 256 + 16 + iota16 + vb, 0)
        m1 = jnp.max(keyb)
        m2 = jnp.max(jnp.where(keyb == m1, 0, keyb))
        l1n = jnp.where(l1 == 0, m1, l1)
        l2n = jnp.where(l1 == 0, m2, jnp.where(l2 == 0, m1, l2))
        return (l1n, l2n, cb - 1)

    l1, l2, _ = lax.while_loop(
        bs_cond, bs_body,
        (jnp.int32(0), jnp.int32(0),
         jnp.where(h == 1, NCH - 1, -1).astype(jnp.int32)))

    @pl.when(h == 1)
    def _():
        w = buf[pl.ds(PAD + SEG - CH, CH)]
        buf[pl.ds(PAD + SEG - CH, CH)] = jnp.where(iota == 15, 0.0, w)

    ebase = h * SEG
    kbase = ebase * 16
    init = (l1, l2, jnp.int32(0), zi, zi)

    @plsc.parallel_loop(0, NCH, 1, unroll=2, carry=init)
    def fin(c, carry):
        carryE, carryO, cpar, acc1, acc2 = carry
        base = PAD + ebase + c * CH
        xc = buf[pl.ds(base, CH)]
        xp = buf[pl.ds(base - 1, CH)]
        vp = xp.astype(jnp.int32)
        mp = vp != 0
        mpi = jnp.where(mp, 1, 0)
        cs = plsc.cumsum(mpi)
        rank = cs + cpar
        par_even = (rank & 1) == 0
        kraw = kbase + c * 256 + iota16 + vp
        keyp = jnp.where(mp, kraw, 0)
        keyE = jnp.where(par_even, keyp, zi)
        keyO = jnp.where(par_even, zi, keyp)
        cumE = plsc.cummax(keyE)
        cumO = plsc.cummax(keyO)
        ME = jnp.maximum(cumE, carryE)
        MO = jnp.maximum(cumO, carryO)
        p1k = jnp.maximum(ME, MO)
        p2k = jnp.minimum(ME, MO)
        vc = xc.astype(jnp.int32)
        mc = xc != 0.0
        mykey = kbase + c * 256 + 16 + iota16 + vc
        d1 = (mykey - p1k) & 15
        d2 = (mykey - p2k) & 15
        a1 = mc & (p1k > 0)
        t1 = plsc.load_gather(tab, [d1])
        acc1 = acc1 + jnp.where(a1, t1, 0)
        hit2 = mc & (p2k > 0) & (d2 == 0)
        acc2 = acc2 + jnp.where(hit2, 1, 0) + jnp.where(mc, 1 << 16, 0)
        return (jnp.maximum(carryE, cumE[15]),
                jnp.maximum(carryO, cumO[15]),
                (cpar + cs[15]) & 1,
                acc1, acc2)

    stage[pl.ds(0, CH)] = fin[3]
    stage[pl.ds(CH, CH)] = fin[4]
    pltpu.sync_copy(stage, shared.at[s])
    plsc.subcore_barrier()

    @pl.when(h == 0)
    def _():
        pltpu.sync_copy(shared.at[s + 1], stage2)
        acc1 = fin[3] + stage2[pl.ds(0, CH)]
        acc2 = fin[4] + stage2[pl.ds(CH, CH)]
        rep = jnp.sum(acc1 & 1023).astype(jnp.float32)
        inc = jnp.sum((acc1 >> 10) & 1023).astype(jnp.float32)
        dec = jnp.sum(acc1 >> 20).astype(jnp.float32)
        p2 = jnp.sum(acc2 & 0xFFFF).astype(jnp.float32)
        cnt = jnp.sum(acc2 >> 16)
        cf = cnt.astype(jnp.float32)
        den1 = jnp.maximum(cf - 1.0, 1.0)
        den2 = jnp.maximum(cf - 2.0, 1.0)
        num = jnp.where(iota == 0, rep,
              jnp.where(iota == 1, inc,
              jnp.where(iota == 2, dec,
              jnp.where(iota == 3, p2, 0.0))))
        den = jnp.where(iota == 3, den2, den1)
        gate = jnp.where(iota < 3, cnt > 1, cnt >= 4) & (iota < 4)
        res[pl.ds(0, CH)] = jnp.where(gate, num / den, 0.0)
        pltpu.sync_copy(res, out_hbm.at[row])


@jax.jit
def kernel(x):
    run = pl.kernel(
        _body,
        out_type=jax.ShapeDtypeStruct((B, OUTW), jnp.float32),
        mesh=plsc.VectorSubcoreMesh(core_axis_name="c", subcore_axis_name="s"),
        scratch_types=[
            pltpu.VMEM((PAD + L,), jnp.float32),
            pltpu.VMEM((CH,), jnp.int32),
            pltpu.VMEM((2 * CH,), jnp.int32),
            pltpu.VMEM((2 * CH,), jnp.int32),
            pltpu.VMEM((OUTW,), jnp.float32),
            pltpu.VMEM_SHARED((16, 2 * CH), jnp.int32),
        ],
        compiler_params=pltpu.CompilerParams(
            needs_layout_passes=False, use_tc_tiling_on_sc=False),
    )
    return run(x)[:, :4]


# speedup vs baseline: 1.0778x; 1.0022x over previous
import jax
import jax.numpy as jnp
from jax import lax
from jax.experimental import pallas as pl
from jax.experimental.pallas import tpu as pltpu
from jax.experimental.pallas import tpu_sc as plsc

B = 16
L = 4096
CH = 16
SEG = L // 2
NCH = SEG // CH
PAD = 8
OUTW = 16


def _body(x_hbm, out_hbm, buf, tab, stage, stage2, res, shared):
    cc = lax.axis_index("c")
    s = lax.axis_index("s")
    row = cc * 8 + (s // 2)
    h = s & 1

    zeros16 = jnp.zeros((CH,), jnp.float32)
    buf[pl.ds(0, CH)] = zeros16
    iota = lax.iota(jnp.int32, CH)
    iota16 = iota * 16
    zi = jnp.zeros((CH,), jnp.int32)
    tab[pl.ds(0, CH)] = jnp.where(iota == 0, 1,
                        jnp.where(iota <= 6, 1 << 10,
                        jnp.where(iota >= 10, 1 << 20, 0)))

    @pl.when(h == 0)
    def _():
        pltpu.sync_copy(x_hbm.at[row, pl.ds(0, SEG)], buf.at[pl.ds(PAD, SEG)])

    @pl.when(h == 1)
    def _():
        pltpu.sync_copy(x_hbm.at[row, pl.ds(SEG - CH, SEG + CH)],
                        buf.at[pl.ds(PAD + SEG - CH, SEG + CH)])

    def bs_cond(st):
        return (st[1] == 0) & (st[2] >= 0)

    def bs_body(st):
        l1, l2, cb = st

        @pl.when(cb < NCH - 1)
        def _():
            pltpu.sync_copy(x_hbm.at[row, pl.ds(cb * CH, CH)],
                            buf.at[pl.ds(PAD + cb * CH, CH)])

        xb = buf[pl.ds(PAD + cb * CH, CH)]
        vb = xb.astype(jnp.int32)
        keyb = jnp.where(vb != 0, cb * 256 + 16 + iota16 + vb, 0)
        m1 = jnp.max(keyb)
        m2 = jnp.max(jnp.where(keyb == m1, 0, keyb))
        l1n = jnp.where(l1 == 0, m1, l1)
        l2n = jnp.where(l1 == 0, m2, jnp.where(l2 == 0, m1, l2))
        return (l1n, l2n, cb - 1)

    l1, l2, _ = lax.while_loop(
        bs_cond, bs_body,
        (jnp.int32(0), jnp.int32(0),
         jnp.where(h == 1, NCH - 1, -1).astype(jnp.int32)))

    @pl.when(h == 1)
    def _():
        w = buf[pl.ds(PAD + SEG - CH, CH)]
        buf[pl.ds(PAD + SEG - CH, CH)] = jnp.where(iota == 15, 0.0, w)

    ebase = h * SEG
    kbase = ebase * 16
    init = (l1, l2, jnp.int32(0), zi, zi)

    @plsc.parallel_loop(0, NCH, 1, unroll=2, carry=init)
    def fin(c, carry):
        carryE, carryO, cpar, acc1, acc2 = carry
        base = PAD + ebase + c * CH
        xc = buf[pl.ds(base, CH)]
        xp = buf[pl.ds(base - 1, CH)]
        vp = xp.astype(jnp.int32)
        mp = vp != 0
        mpi = jnp.where(mp, 1, 0)
        cs = plsc.cumsum(mpi)
        rank = cs + cpar
        par_even = (rank & 1) == 0
        kraw = kbase + c * 256 + iota16 + vp
        keyp = jnp.where(mp, kraw, 0)
        keyE = jnp.where(par_even, keyp, zi)
        keyO = jnp.where(par_even, zi, keyp)
        cumE = plsc.cummax(keyE)
        cumO = plsc.cummax(keyO)
        ME = jnp.maximum(cumE, carryE)
        MO = jnp.maximum(cumO, carryO)
        p1k = jnp.maximum(ME, MO)
        p2k = jnp.minimum(ME, MO)
        vc = xc.astype(jnp.int32)
        mc = xc != 0.0
        mykey = kbase + c * 256 + 16 + iota16 + vc
        d1 = (mykey - p1k) & 15
        d2 = (mykey - p2k) & 15
        a1 = mc & (p1k > 0)
        t1 = plsc.load_gather(tab, [d1])
        acc1 = acc1 + jnp.where(a1, t1, 0)
        hit2 = mc & (p2k > 0) & (d2 == 0)
        acc2 = acc2 + jnp.where(hit2, 1, 0) + jnp.where(mc, 1 << 16, 0)
        return (jnp.maximum(carryE, cumE[15]),
                jnp.maximum(carryO, cumO[15]),
                (cpar + cs[15]) & 1,
                acc1, acc2)

    stage[pl.ds(0, CH)] = fin[3]
    stage[pl.ds(CH, CH)] = fin[4]
    pltpu.sync_copy(stage, shared.at[s])
    plsc.subcore_barrier()

    @pl.when(h == 0)
    def _():
        pltpu.sync_copy(shared.at[s + 1], stage2)
        acc1 = fin[3] + stage2[pl.ds(0, CH)]
        acc2 = fin[4] + stage2[pl.ds(CH, CH)]
        rep = jnp.sum(acc1 & 1023).astype(jnp.float32)
        inc = jnp.sum((acc1 >> 10) & 1023).astype(jnp.float32)
        dec = jnp.sum(acc1 >> 20).astype(jnp.float32)
        p2 = jnp.sum(acc2 & 0xFFFF).astype(jnp.float32)
        cnt = jnp.sum(acc2 >> 16)
        cf = cnt.astype(jnp.float32)
        den1 = jnp.maximum(cf - 1.0, 1.0)
        den2 = jnp.maximum(cf - 2.0, 1.0)
        num = jnp.where(iota == 0, rep,
              jnp.where(iota == 1, inc,
              jnp.where(iota == 2, dec,
              jnp.where(iota == 3, p2, 0.0))))
        den = jnp.where(iota == 3, den2, den1)
        gate = jnp.where(iota < 3, cnt > 1, cnt >= 4) & (iota < 4)
        res[pl.ds(0, CH)] = jnp.where(gate, num / den, 0.0)
        pltpu.sync_copy(res, out_hbm.at[row])


@jax.jit
def kernel(x):
    run = pl.kernel(
        _body,
        out_type=jax.ShapeDtypeStruct((B, OUTW), jnp.float32),
        mesh=plsc.VectorSubcoreMesh(core_axis_name="c", subcore_axis_name="s"),
        scratch_types=[
            pltpu.VMEM((PAD + L,), jnp.float32),
            pltpu.VMEM((CH,), jnp.int32),
            pltpu.VMEM((2 * CH,), jnp.int32),
            pltpu.VMEM((2 * CH,), jnp.int32),
            pltpu.VMEM((OUTW,), jnp.float32),
            pltpu.VMEM_SHARED((16, 2 * CH), jnp.int32),
        ],
        compiler_params=pltpu.CompilerParams(
            needs_layout_passes=False, use_tc_tiling_on_sc=False),
    )
    return run(x)[:, :4]


# speedup vs baseline: 1.0943x; 1.0153x over previous
import jax
import jax.numpy as jnp
from jax import lax
from jax.experimental import pallas as pl
from jax.experimental.pallas import tpu as pltpu
from jax.experimental.pallas import tpu_sc as plsc

B = 16
L = 4096
CH = 16
SEG = L // 2
NCH = SEG // CH
PAD = 8
OUTW = 16


def _body(x_hbm, out_hbm, buf, tab, tab2, ct, stage, stage2, res, shared):
    cc = lax.axis_index("c")
    s = lax.axis_index("s")
    row = cc * 8 + (s // 2)
    h = s & 1

    zeros16 = jnp.zeros((CH,), jnp.float32)
    buf[pl.ds(0, CH)] = zeros16
    iota = lax.iota(jnp.int32, CH)
    iota16 = iota * 16
    zi = jnp.zeros((CH,), jnp.int32)
    iota64 = iota * 64
    ct[pl.ds(0, CH)] = jnp.where(iota == 0, 15, 2 * iota + 16)
    for g in range(4):
        dv = g * 16 + iota
        ev = (dv & 1) == 0
        tab[pl.ds(g * CH, CH)] = jnp.where(
            dv == 0, 1,
            jnp.where(ev & (dv >= 2) & (dv <= 12), 1 << 10,
            jnp.where(ev & (dv >= 52), 1 << 20, 0)))
        tab2[pl.ds(g * CH, CH)] = (jnp.where(ev, 1 << 16, 0)
                                   + jnp.where(dv == 0, 1, 0))

    @pl.when(h == 0)
    def _():
        pltpu.sync_copy(x_hbm.at[row, pl.ds(0, SEG)], buf.at[pl.ds(PAD, SEG)])

    @pl.when(h == 1)
    def _():
        pltpu.sync_copy(x_hbm.at[row, pl.ds(SEG - CH, SEG + CH)],
                        buf.at[pl.ds(PAD + SEG - CH, SEG + CH)])

    def bs_cond(st):
        return (st[1] == 0) & (st[2] >= 0)

    def bs_body(st):
        l1, l2, cb = st

        @pl.when(cb < NCH - 1)
        def _():
            pltpu.sync_copy(x_hbm.at[row, pl.ds(cb * CH, CH)],
                            buf.at[pl.ds(PAD + cb * CH, CH)])

        xb = buf[pl.ds(PAD + cb * CH, CH)]
        vb = xb.astype(jnp.int32)
        keyb = jnp.where(vb != 0,
                         cb * 1024 + 64 + iota64 + plsc.load_gather(ct, [vb]),
                         0)
        m1 = jnp.max(keyb)
        m2 = jnp.max(jnp.where(keyb == m1, 0, keyb))
        l1n = jnp.where(l1 == 0, m1, l1)
        l2n = jnp.where(l1 == 0, m2, jnp.where(l2 == 0, m1, l2))
        return (l1n, l2n, cb - 1)

    l1, l2, _ = lax.while_loop(
        bs_cond, bs_body,
        (jnp.int32(0), jnp.int32(0),
         jnp.where(h == 1, NCH - 1, -1).astype(jnp.int32)))

    @pl.when(h == 1)
    def _():
        w = buf[pl.ds(PAD + SEG - CH, CH)]
        buf[pl.ds(PAD + SEG - CH, CH)] = jnp.where(iota == 15, 0.0, w)

    ebase = h * SEG
    kbase = ebase * 64
    init = (l1, l2, jnp.int32(0), zi, zi)

    @plsc.parallel_loop(0, NCH, 1, unroll=2, carry=init)
    def fin(c, carry):
        carryE, carryO, cpar, acc1, acc2 = carry
        base = PAD + ebase + c * CH
        xc = buf[pl.ds(base, CH)]
        xp = buf[pl.ds(base - 1, CH)]
        vp = xp.astype(jnp.int32)
        mp = vp != 0
        mpi = jnp.where(mp, 1, 0)
        cs = plsc.cumsum(mpi)
        rank = cs + cpar
        par_even = (rank & 1) == 0
        kraw = kbase + c * 1024 + iota64 + plsc.load_gather(ct, [vp])
        keyp = jnp.where(mp, kraw, 0)
        keyE = jnp.where(par_even, keyp, zi)
        keyO = jnp.where(par_even, zi, keyp)
        cumE = plsc.cummax(keyE)
        cumO = plsc.cummax(keyO)
        ME = jnp.maximum(cumE, carryE)
        MO = jnp.maximum(cumO, carryO)
        p1k = jnp.maximum(ME, MO)
        p2k = jnp.minimum(ME, MO)
        vc = xc.astype(jnp.int32)
        mykey = kbase + c * 1024 + 64 + iota64 + plsc.load_gather(ct, [vc])
        d1 = (mykey - p1k) & 63
        d2 = (mykey - p2k) & 63
        acc1 = acc1 + plsc.load_gather(tab, [d1])
        acc2 = acc2 + plsc.load_gather(tab2, [d2])
        return (jnp.maximum(carryE, cumE[15]),
                jnp.maximum(carryO, cumO[15]),
                (cpar + cs[15]) & 1,
                acc1, acc2)

    stage[pl.ds(0, CH)] = fin[3]
    stage[pl.ds(CH, CH)] = fin[4]
    pltpu.sync_copy(stage, shared.at[s])
    plsc.subcore_barrier()

    @pl.when(h == 0)
    def _():
        pltpu.sync_copy(shared.at[s + 1], stage2)
        acc1 = fin[3] + stage2[pl.ds(0, CH)]
        acc2 = fin[4] + stage2[pl.ds(CH, CH)]
        rep = jnp.sum(acc1 & 1023).astype(jnp.float32)
        inc = jnp.sum((acc1 >> 10) & 1023).astype(jnp.float32)
        dec = jnp.sum(acc1 >> 20).astype(jnp.float32)
        p2 = jnp.sum(acc2 & 0xFFFF).astype(jnp.float32)
        cnt = jnp.sum(acc2 >> 16)
        cf = cnt.astype(jnp.float32)
        den1 = jnp.maximum(cf - 1.0, 1.0)
        den2 = jnp.maximum(cf - 2.0, 1.0)
        num = jnp.where(iota == 0, rep,
              jnp.where(iota == 1, inc,
              jnp.where(iota == 2, dec,
              jnp.where(iota == 3, p2, 0.0))))
        den = jnp.where(iota == 3, den2, den1)
        gate = jnp.where(iota < 3, cnt > 1, cnt >= 4) & (iota < 4)
        res[pl.ds(0, CH)] = jnp.where(gate, num / den, 0.0)
        pltpu.sync_copy(res, out_hbm.at[row])


@jax.jit
def kernel(x):
    run = pl.kernel(
        _body,
        out_type=jax.ShapeDtypeStruct((B, OUTW), jnp.float32),
        mesh=plsc.VectorSubcoreMesh(core_axis_name="c", subcore_axis_name="s"),
        scratch_types=[
            pltpu.VMEM((PAD + L,), jnp.float32),
            pltpu.VMEM((4 * CH,), jnp.int32),
            pltpu.VMEM((4 * CH,), jnp.int32),
            pltpu.VMEM((CH,), jnp.int32),
            pltpu.VMEM((2 * CH,), jnp.int32),
            pltpu.VMEM((2 * CH,), jnp.int32),
            pltpu.VMEM((OUTW,), jnp.float32),
            pltpu.VMEM_SHARED((16, 2 * CH), jnp.int32),
        ],
        compiler_params=pltpu.CompilerParams(
            needs_layout_passes=False, use_tc_tiling_on_sc=False),
    )
    return run(x)[:, :4]
